# baseline (speedup 1.0000x reference)
import functools

import jax
import jax.numpy as jnp
from jax.experimental import pallas as pl
from jax.experimental.pallas import tpu as pltpu
from jax.experimental.pallas import tpu_sc as plsc

N = 50000
D = 512
S = 64
B = 2000
NB = N // B
R2 = 5
NB2 = NB // R2

_NEG_INF = float("-inf")


def _sweep_kernel(x_ref, browp_ref, w1_ref, b1_ref,
                  w2_ref,
                  scores_ref, out_ref, q2_ref, qf_ref,
                  m_ref, d_ref, o_ref, sprev_ref, xbprev_ref):
    i = pl.program_id(0)

    @pl.when(i == 0)
    def _init():
        m_ref[...] = jnp.full_like(m_ref, _NEG_INF)
        d_ref[...] = jnp.zeros_like(d_ref)
        o_ref[...] = jnp.zeros_like(o_ref)

    x = x_ref[...]
    xb = x.astype(jnp.bfloat16)
    h = jnp.tanh(
        jnp.dot(xb, w1_ref[...], preferred_element_type=jnp.float32)
        + b1_ref[...])
    s = jnp.dot(h.astype(jnp.bfloat16), w2_ref[...],
                preferred_element_type=jnp.float32)

    s_prev = sprev_ref[...]
    browp = browp_ref[...].reshape(1, B)
    bcolp = browp.reshape(B, 1)
    mask = bcolp == jax.lax.broadcasted_iota(jnp.int32, (B, S), 1)
    mask_b = mask.astype(jnp.bfloat16)
    mask_t_b = (browp == jax.lax.broadcasted_iota(jnp.int32, (S, B), 0)
                ).astype(jnp.bfloat16)

    first = i == 0
    sm = jnp.where(first, _NEG_INF,
                   jnp.max(jnp.where(mask, s_prev, _NEG_INF), axis=0,
                           keepdims=True))
    m_old = m_ref[...]
    m_new = jnp.maximum(m_old, sm).astype(jnp.bfloat16).astype(jnp.float32)
    m_safe = jnp.where(m_new == _NEG_INF, 0.0, m_new)
    r = jnp.where(m_old == _NEG_INF, 0.0, jnp.exp(m_old - m_safe))
    r_col = r.reshape(S, 1)
    m_ref[...] = m_new

    mg = jnp.dot(mask_b, m_safe.reshape(S, 1).astype(jnp.bfloat16),
                 preferred_element_type=jnp.float32)
    ex = jnp.exp(s_prev - mg)
    exb = ex.astype(jnp.bfloat16)
    dsum = jnp.dot(mask_t_b, exb,
                   preferred_element_type=jnp.float32)
    d_ref[...] = d_ref[...] * r_col + jnp.where(first, 0.0, dsum)
    xw = xbprev_ref[...] * exb
    po = jnp.dot(mask_t_b, xw,
                 preferred_element_type=jnp.float32)
    o_ref[...] = o_ref[...] * r_col + jnp.where(first, 0.0, po)

    scores_ref[...] = s.reshape(1, 1, B)
    sprev_ref[...] = s
    xbprev_ref[...] = xb

    @pl.when(i == NB)
    def _fin():
        d = d_ref[...]
        out_ref[...] = o_ref[...] * (1.0 / (d + 1e-16))
        m_fin = jnp.where(m_ref[...] == _NEG_INF, 0.0, m_ref[...])
        q = m_fin.reshape(S, 1) + jnp.log(d + 1e-16)
        qhi = q.astype(jnp.bfloat16)
        qlo = (q - qhi.astype(jnp.float32)).astype(jnp.bfloat16)
        q2_ref[...] = jnp.concatenate(
            [qhi.reshape(1, S), qlo.reshape(1, S)], axis=0)
        qf_ref[...] = q.reshape(1, S)


_SC_NC = 2
_SC_NS = 16
_SC_NW = _SC_NC * _SC_NS
_NPAD = 50176
_SC_CHUNK = _NPAD // _SC_NW


def _sc_weights_body(scores_hbm, batch_hbm, q_hbm, w_hbm,
                     idx_v, s_v, qg_v, w_v, sem):
    wid = jax.lax.axis_index("s") * _SC_NC + jax.lax.axis_index("c")
    base = wid * _SC_CHUNK
    pltpu.sync_copy(batch_hbm.at[pl.ds(base, _SC_CHUNK)], idx_v)
    pltpu.async_copy(q_hbm.at[idx_v], qg_v, sem).wait()
    pltpu.sync_copy(scores_hbm.at[pl.ds(base, _SC_CHUNK)], s_v)

    def body(k, carry):
        sl = pl.ds(k * 16, 16)
        w_v[sl] = jnp.exp(s_v[sl] - qg_v[sl])
        return carry

    jax.lax.fori_loop(0, _SC_CHUNK // 16, body, 0)
    pltpu.sync_copy(w_v, w_hbm.at[pl.ds(base, _SC_CHUNK)])


def _sc_weights(scores_pad, batch_pad, q_vec):
    mesh = plsc.VectorSubcoreMesh(core_axis_name="c", subcore_axis_name="s")
    k = functools.partial(
        pl.kernel, mesh=mesh,
        out_type=jax.ShapeDtypeStruct((_NPAD,), jnp.float32),
        scratch_types=[
            pltpu.VMEM((_SC_CHUNK,), jnp.int32),
            pltpu.VMEM((_SC_CHUNK,), jnp.float32),
            pltpu.VMEM((_SC_CHUNK,), jnp.float32),
            pltpu.VMEM((_SC_CHUNK,), jnp.float32),
            pltpu.SemaphoreType.DMA,
        ],
    )(_sc_weights_body)
    return k(scores_pad, batch_pad, q_vec)


def _weights_kernel(scores_ref, brow_ref, q2_ref, w_ref):
    q2 = q2_ref[...]
    for r in range(R2):
        srow = scores_ref[r]
        brow = brow_ref[r]
        mask_t_b = (brow == jax.lax.broadcasted_iota(jnp.int32, (S, B), 0)
                    ).astype(jnp.bfloat16)
        mg2 = jnp.dot(q2, mask_t_b,
                      preferred_element_type=jnp.float32)
        w_ref[r] = jnp.exp(srow - mg2[0:1, :] - mg2[1:2, :])


def kernel(x, batch, W1, b1, W2, b2):
    brow3 = batch.astype(jnp.int32).reshape(NB, 1, B)
    w1b = W1.astype(jnp.bfloat16)
    w2b = W2.astype(jnp.bfloat16)
    b1r = b1.reshape(1, D)

    _clamp = lambda i: jnp.minimum(i, NB - 1)
    _prev = lambda i: jnp.clip(i - 1, 0, NB - 1)
    scores, out, q2, qf = pl.pallas_call(
        _sweep_kernel,
        grid=(NB + 1,),
        in_specs=[
            pl.BlockSpec((B, D), lambda i: (_clamp(i), 0)),
            pl.BlockSpec((1, 1, B), lambda i: (_prev(i), 0, 0)),
            pl.BlockSpec((D, D), lambda i: (0, 0)),
            pl.BlockSpec((1, D), lambda i: (0, 0)),
            pl.BlockSpec((D, 1), lambda i: (0, 0)),
        ],
        out_specs=[
            pl.BlockSpec((1, 1, B), lambda i: (_clamp(i), 0, 0)),
            pl.BlockSpec((S, D), lambda i: (0, 0)),
            pl.BlockSpec((2, S), lambda i: (0, 0)),
            pl.BlockSpec((1, S), lambda i: (0, 0)),
        ],
        out_shape=[
            jax.ShapeDtypeStruct((NB, 1, B), jnp.float32),
            jax.ShapeDtypeStruct((S, D), jnp.float32),
            jax.ShapeDtypeStruct((2, S), jnp.bfloat16),
            jax.ShapeDtypeStruct((1, S), jnp.float32),
        ],
        scratch_shapes=[
            pltpu.VMEM((1, S), jnp.float32),
            pltpu.VMEM((S, 1), jnp.float32),
            pltpu.VMEM((S, D), jnp.float32),
            pltpu.VMEM((B, 1), jnp.float32),
            pltpu.VMEM((B, D), jnp.bfloat16),
        ],
        compiler_params=pltpu.CompilerParams(
            dimension_semantics=("arbitrary",)),
    )(x, brow3, w1b, b1r, w2b)

    scores_pad = jnp.pad(scores.reshape(N), (0, _NPAD - N))
    batch_pad = jnp.pad(batch.astype(jnp.int32), (0, _NPAD - N))
    w_pad = _sc_weights(scores_pad, batch_pad, qf.reshape(S))

    return out, w_pad[:N]


# speedup vs baseline: 3.5746x; 3.5746x over previous
import jax
import jax.numpy as jnp
from jax.experimental import pallas as pl
from jax.experimental.pallas import tpu as pltpu

N = 50000
D = 512
S = 64
B = 2000
NB = N // B
R2 = 5
NB2 = NB // R2

_NEG_INF = float("-inf")


def _sweep_kernel(x_ref, browp_ref, w1_ref, b1_ref,
                  w2_ref,
                  scores_ref, out_ref, q2_ref,
                  m_ref, d_ref, o_ref, sprev_ref, xbprev_ref):
    i = pl.program_id(0)

    @pl.when(i == 0)
    def _init():
        m_ref[...] = jnp.full_like(m_ref, _NEG_INF)
        d_ref[...] = jnp.zeros_like(d_ref)
        o_ref[...] = jnp.zeros_like(o_ref)

    x = x_ref[...]
    xb = x.astype(jnp.bfloat16)
    h = jnp.tanh(
        jnp.dot(xb, w1_ref[...], preferred_element_type=jnp.float32)
        + b1_ref[...])
    s = jnp.dot(h.astype(jnp.bfloat16), w2_ref[...],
                preferred_element_type=jnp.float32)

    s_prev = sprev_ref[...]
    browp = browp_ref[...].reshape(1, B)
    bcolp = browp.reshape(B, 1)
    mask = bcolp == jax.lax.broadcasted_iota(jnp.int32, (B, S), 1)
    mask_b = mask.astype(jnp.bfloat16)
    mask_t_b = (browp == jax.lax.broadcasted_iota(jnp.int32, (S, B), 0)
                ).astype(jnp.bfloat16)

    first = i == 0
    sm = jnp.where(first, _NEG_INF,
                   jnp.max(jnp.where(mask, s_prev, _NEG_INF), axis=0,
                           keepdims=True))
    m_old = m_ref[...]
    m_new = jnp.maximum(m_old, sm).astype(jnp.bfloat16).astype(jnp.float32)
    m_safe = jnp.where(m_new == _NEG_INF, 0.0, m_new)
    r = jnp.where(m_old == _NEG_INF, 0.0, jnp.exp(m_old - m_safe))
    r_col = r.reshape(S, 1)
    m_ref[...] = m_new

    mg = jnp.dot(mask_b, m_safe.reshape(S, 1).astype(jnp.bfloat16),
                 preferred_element_type=jnp.float32)
    ex = jnp.exp(s_prev - mg)
    exb = ex.astype(jnp.bfloat16)
    dsum = jnp.dot(mask_t_b, exb,
                   preferred_element_type=jnp.float32)
    d_ref[...] = d_ref[...] * r_col + jnp.where(first, 0.0, dsum)
    xw = xbprev_ref[...] * exb
    po = jnp.dot(mask_t_b, xw,
                 preferred_element_type=jnp.float32)
    o_ref[...] = o_ref[...] * r_col + jnp.where(first, 0.0, po)

    scores_ref[...] = s.reshape(1, 1, B)
    sprev_ref[...] = s
    xbprev_ref[...] = xb

    @pl.when(i == NB)
    def _fin():
        d = d_ref[...]
        out_ref[...] = o_ref[...] * (1.0 / (d + 1e-16))
        m_fin = jnp.where(m_ref[...] == _NEG_INF, 0.0, m_ref[...])
        q = m_fin.reshape(S, 1) + jnp.log(d + 1e-16)
        qhi = q.astype(jnp.bfloat16)
        qlo = (q - qhi.astype(jnp.float32)).astype(jnp.bfloat16)
        q2_ref[...] = jnp.concatenate(
            [qhi.reshape(1, S), qlo.reshape(1, S)], axis=0)


def _weights_kernel(scores_ref, brow_ref, q2_ref, w_ref):
    q2 = q2_ref[...]
    for r in range(R2):
        srow = scores_ref[r]
        brow = brow_ref[r]
        mask_t_b = (brow == jax.lax.broadcasted_iota(jnp.int32, (S, B), 0)
                    ).astype(jnp.bfloat16)
        mg2 = jnp.dot(q2, mask_t_b,
                      preferred_element_type=jnp.float32)
        w_ref[r] = jnp.exp(srow - mg2[0:1, :] - mg2[1:2, :])


def kernel(x, batch, W1, b1, W2, b2):
    brow3 = batch.astype(jnp.int32).reshape(NB, 1, B)
    w1b = W1.astype(jnp.bfloat16)
    w2b = W2.astype(jnp.bfloat16)
    b1r = b1.reshape(1, D)

    _clamp = lambda i: jnp.minimum(i, NB - 1)
    _prev = lambda i: jnp.clip(i - 1, 0, NB - 1)
    scores, out, q2 = pl.pallas_call(
        _sweep_kernel,
        grid=(NB + 1,),
        in_specs=[
            pl.BlockSpec((B, D), lambda i: (_clamp(i), 0)),
            pl.BlockSpec((1, 1, B), lambda i: (_prev(i), 0, 0)),
            pl.BlockSpec((D, D), lambda i: (0, 0)),
            pl.BlockSpec((1, D), lambda i: (0, 0)),
            pl.BlockSpec((D, 1), lambda i: (0, 0)),
        ],
        out_specs=[
            pl.BlockSpec((1, 1, B), lambda i: (_clamp(i), 0, 0)),
            pl.BlockSpec((S, D), lambda i: (0, 0)),
            pl.BlockSpec((2, S), lambda i: (0, 0)),
        ],
        out_shape=[
            jax.ShapeDtypeStruct((NB, 1, B), jnp.float32),
            jax.ShapeDtypeStruct((S, D), jnp.float32),
            jax.ShapeDtypeStruct((2, S), jnp.bfloat16),
        ],
        scratch_shapes=[
            pltpu.VMEM((1, S), jnp.float32),
            pltpu.VMEM((S, 1), jnp.float32),
            pltpu.VMEM((S, D), jnp.float32),
            pltpu.VMEM((B, 1), jnp.float32),
            pltpu.VMEM((B, D), jnp.bfloat16),
        ],
        compiler_params=pltpu.CompilerParams(
            dimension_semantics=("arbitrary",)),
    )(x, brow3, w1b, b1r, w2b)

    scores3 = scores
    w3 = pl.pallas_call(
        _weights_kernel,
        grid=(NB2,),
        in_specs=[
            pl.BlockSpec((R2, 1, B), lambda i: (i, 0, 0)),
            pl.BlockSpec((R2, 1, B), lambda i: (i, 0, 0)),
            pl.BlockSpec((2, S), lambda i: (0, 0)),
        ],
        out_specs=pl.BlockSpec((R2, 1, B), lambda i: (i, 0, 0)),
        out_shape=jax.ShapeDtypeStruct((NB, 1, B), jnp.float32),
        compiler_params=pltpu.CompilerParams(
            dimension_semantics=("arbitrary",)),
    )(scores3, brow3, q2)

    return out, w3.reshape(N)
